# baseline (speedup 1.0000x reference)
import functools

import jax
import jax.numpy as jnp
from jax import lax
from jax.experimental import pallas as pl
from jax.experimental.pallas import tpu as pltpu
from jax.experimental.pallas import tpu_sc as plsc

NC = 2
NS = 16
NW = NC * NS
CHUNK = 256


def kernel(source, W):
    seq, batch, _ = source.shape
    dim = W.shape[1]
    B = seq * batch
    n_chunks = B // (NW * CHUNK)
    ids_per_w = n_chunks * CHUNK
    assert n_chunks * NW * CHUNK == B

    idx = source.reshape(B)
    mesh = plsc.VectorSubcoreMesh(core_axis_name="c", subcore_axis_name="s")

    @functools.partial(
        pl.kernel,
        out_type=jax.ShapeDtypeStruct((NW, n_chunks, CHUNK, dim), jnp.float32),
        mesh=mesh,
        scratch_types=[
            pltpu.VMEM((ids_per_w,), jnp.int32),
            pltpu.VMEM((n_chunks, CHUNK, dim), jnp.float32),
            pltpu.SemaphoreType.DMA,
            pltpu.SemaphoreType.DMA,
            pltpu.SemaphoreType.DMA,
        ],
    )
    def gather_kernel(table_hbm, idx_hbm, out_hbm, idx_v, rows_v,
                      sem_i, sem_g, sem_w):
        wid = lax.axis_index("s") * NC + lax.axis_index("c")
        idx_copies = [
            pltpu.async_copy(
                idx_hbm.at[pl.ds(wid * ids_per_w + j * CHUNK, CHUNK)],
                idx_v.at[pl.ds(j * CHUNK, CHUNK)],
                sem_i,
            )
            for j in range(n_chunks)
        ]
        gathers = []
        for j in range(n_chunks):
            idx_copies[j].wait()
            gathers.append(
                pltpu.async_copy(
                    table_hbm.at[idx_v.at[pl.ds(j * CHUNK, CHUNK)]],
                    rows_v.at[j],
                    sem_g,
                )
            )
        writes = []
        for j in range(n_chunks):
            gathers[j].wait()
            writes.append(
                pltpu.async_copy(rows_v.at[j], out_hbm.at[wid, j], sem_w)
            )
        for w in writes:
            w.wait()

    out = gather_kernel(W, idx)
    return out.reshape(seq, batch, dim)


# speedup vs baseline: 1.0111x; 1.0111x over previous
import functools

import jax
import jax.numpy as jnp
from jax import lax
from jax.experimental import pallas as pl
from jax.experimental.pallas import tpu as pltpu
from jax.experimental.pallas import tpu_sc as plsc

NC = 2
NS = 16
NW = NC * NS
CHUNK = 128


def kernel(source, W):
    seq, batch, _ = source.shape
    dim = W.shape[1]
    B = seq * batch
    n_chunks = B // (NW * CHUNK)
    ids_per_w = n_chunks * CHUNK
    assert n_chunks * NW * CHUNK == B

    idx = source.reshape(B)
    mesh = plsc.VectorSubcoreMesh(core_axis_name="c", subcore_axis_name="s")

    @functools.partial(
        pl.kernel,
        out_type=jax.ShapeDtypeStruct((NW, n_chunks, CHUNK, dim), jnp.float32),
        mesh=mesh,
        scratch_types=[
            pltpu.VMEM((ids_per_w,), jnp.int32),
            pltpu.VMEM((n_chunks, CHUNK, dim), jnp.float32),
            pltpu.SemaphoreType.DMA,
            pltpu.SemaphoreType.DMA,
            pltpu.SemaphoreType.DMA,
        ],
    )
    def gather_kernel(table_hbm, idx_hbm, out_hbm, idx_v, rows_v,
                      sem_i, sem_g, sem_w):
        wid = lax.axis_index("s") * NC + lax.axis_index("c")
        idx_copies = [
            pltpu.async_copy(
                idx_hbm.at[pl.ds(wid * ids_per_w + j * CHUNK, CHUNK)],
                idx_v.at[pl.ds(j * CHUNK, CHUNK)],
                sem_i,
            )
            for j in range(n_chunks)
        ]
        gathers = []
        for j in range(n_chunks):
            idx_copies[j].wait()
            gathers.append(
                pltpu.async_copy(
                    table_hbm.at[idx_v.at[pl.ds(j * CHUNK, CHUNK)]],
                    rows_v.at[j],
                    sem_g,
                )
            )
        writes = []
        for j in range(n_chunks):
            gathers[j].wait()
            writes.append(
                pltpu.async_copy(rows_v.at[j], out_hbm.at[wid, j], sem_w)
            )
        for w in writes:
            w.wait()

    out = gather_kernel(W, idx)
    return out.reshape(seq, batch, dim)
